# baseline (speedup 1.0000x reference)
import jax
import jax.numpy as jnp
from jax.experimental import pallas as pl
from jax.experimental.pallas import tpu as pltpu

N = 2048
C = 8
GAMMA = 1.0
TP = 512
NP = N // TP


def _kernel(gem_ref, adj_ref, wsub_ref, winter_ref, pred_ref, tgt_ref,
            mask_ref, out_ref):
    u_sub = jnp.sum(wsub_ref[...], axis=0, keepdims=True)
    u_inter = jnp.sum(winter_ref[...], axis=0, keepdims=True)
    U2 = jnp.concatenate([u_sub, u_inter], axis=0)
    GU = jax.lax.dot_general(gem_ref[...], U2,
                             ((((1,), (1,)), ((), ()))),
                             preferred_element_type=jnp.float32)
    GT = GU.T
    gsub_row = GT[0:1]
    ginter_row = GT[1:2]
    gsub_b = gsub_row.astype(jnp.bfloat16)
    ginter_b = ginter_row.astype(jnp.bfloat16)

    rr = jax.lax.broadcasted_iota(jnp.int32, (TP, TP), 0)
    cc = jax.lax.broadcasted_iota(jnp.int32, (TP, TP), 1)
    eye = rr == cc
    erow = jnp.concatenate(
        [1.0 - jnp.sum(
            jnp.where(eye & (adj_ref[i * TP:(i + 1) * TP,
                                     i * TP:(i + 1) * TP] > 0), 1.0, 0.0),
            axis=0, keepdims=True)
         for i in range(NP)], axis=1)

    tgt = tgt_ref[...]
    maskf = mask_ref[...]
    class_ids = jax.lax.broadcasted_iota(jnp.int32, (C, 1), 0)
    ohT = jnp.where(tgt == class_ids, 1.0, 0.0) * maskf
    ncnt = jnp.sum(ohT, axis=1, keepdims=True)
    inv = jnp.where(ncnt > 0, 1.0 / ncnt, 0.0)
    wvrow = jnp.sum(ohT * inv, axis=0, keepdims=True)
    wvcol = wvrow.T
    predT = pred_ref[...].T
    pprow = jnp.sum(ohT * predT, axis=0, keepdims=True)
    oh = ohT.T

    acc = jnp.zeros((1, 1), jnp.float32)
    for p in range(NP):
        base = p * TP
        A_p = adj_ref[pl.ds(base, TP), :]
        Bsub = A_p * gsub_b
        Binter = A_p * ginter_b
        Bcat = jnp.concatenate([Bsub, Binter], axis=0)
        a_p = jnp.sum(Bsub.astype(jnp.float32), axis=1, keepdims=True)
        wv_p = wvcol[base:base + TP]
        pp_p = pprow[0, base:base + TP].reshape(TP, 1)
        oh_p = oh[base:base + TP]
        PG_row = jnp.dot(oh_p, predT,
                         preferred_element_type=jnp.float32)
        tgt_p = tgt[0, base:base + TP].reshape(TP, 1)

        ell_row = (GAMMA - pp_p + PG_row) ** 2
        neq_row = tgt_p != tgt
        wl_row = jnp.where(neq_row, (wv_p * ell_row) * wvrow, 0.0)

        for q in range(NP):
            qb = q * TP
            A_cols = adj_ref[:, pl.ds(qb, TP)]
            SC = jnp.dot(Bcat, A_cols, preferred_element_type=jnp.float32)
            S = SC[:TP] + Bsub[:, qb:qb + TP].astype(jnp.float32) \
                * erow[0, qb:qb + TP][None, :]
            Cm = SC[TP:]
            ratio = (1.0 + a_p - S) / (1.0 + Cm)
            v = 0.5 - 0.5 * jnp.tanh(0.5 * ratio)
            acc = acc + jnp.sum(wl_row[:, qb:qb + TP] * v).reshape(1, 1)
    out_ref[0] = acc


@jax.jit
def kernel(pred, gem, W_sub, W_inter, W_global, target, mask, adj):
    del W_global
    adj_b = adj.astype(jnp.bfloat16)
    tgt = target.astype(jnp.int32).reshape(1, N)
    maskf = mask.astype(jnp.float32).reshape(1, N)

    loss = pl.pallas_call(
        _kernel,
        out_shape=jax.ShapeDtypeStruct((1, 1, 1), jnp.float32),
    )(gem, adj_b, W_sub, W_inter, pred, tgt, maskf)

    return loss.reshape(1)


# speedup vs baseline: 1.0434x; 1.0434x over previous
import jax
import jax.numpy as jnp
from jax.experimental import pallas as pl
from jax.experimental.pallas import tpu as pltpu

N = 2048
C = 8
GAMMA = 1.0
PT = 512
PNP = N // PT
TP = 512
NP = N // TP


def _prep_kernel(gem_ref, adjd_ref, wsub_ref, winter_ref, pred_ref, tgt_ref,
                 mask_ref, gsub_ref, ginter_ref, erow_ref, wvrow_ref,
                 wvcol_ref, pprow_ref, oh_ref, predT_ref):
    i = pl.program_id(0)
    u_sub = jnp.sum(wsub_ref[...], axis=0, keepdims=True)
    u_inter = jnp.sum(winter_ref[...], axis=0, keepdims=True)
    U2 = jnp.concatenate([u_sub, u_inter], axis=0)
    GU = jax.lax.dot_general(gem_ref[...], U2,
                             ((((1,), (1,)), ((), ()))),
                             preferred_element_type=jnp.float32)
    GT = GU.T
    gsub_ref[...] = GT[0:1]
    ginter_ref[...] = GT[1:2]

    rr = jax.lax.broadcasted_iota(jnp.int32, (PT, PT), 0)
    cc = jax.lax.broadcasted_iota(jnp.int32, (PT, PT), 1)
    dvals = jnp.where((rr == cc) & (adjd_ref[...] > 0), 1.0, 0.0)
    erow_ref[...] = 1.0 - jnp.sum(dvals, axis=0, keepdims=True)

    @pl.when(i == 0)
    def _():
        tgt = tgt_ref[...]
        maskf = mask_ref[...]
        class_ids = jax.lax.broadcasted_iota(jnp.int32, (C, 1), 0)
        ohT = jnp.where(tgt == class_ids, 1.0, 0.0) * maskf
        ncnt = jnp.sum(ohT, axis=1, keepdims=True)
        inv = jnp.where(ncnt > 0, 1.0 / ncnt, 0.0)
        wvrow = jnp.sum(ohT * inv, axis=0, keepdims=True)
        wvrow_ref[...] = wvrow
        wvcol_ref[...] = wvrow.T
        predT = pred_ref[...].T
        predT_ref[...] = predT
        pprow_ref[...] = jnp.sum(ohT * predT, axis=0, keepdims=True)
        oh_ref[...] = ohT.T


def _main_kernel(adj_ref, gsub_ref, ginter_ref, wvrow_ref, wvcol_ref,
                 pprow_ref, oh_ref, predT_ref, erow_ref, tgt_ref, out_ref):
    gsub_b = gsub_ref[...].astype(jnp.bfloat16)
    ginter_b = ginter_ref[...].astype(jnp.bfloat16)

    acc = jnp.zeros((1, 1), jnp.float32)
    for p in range(NP):
        base = p * TP
        A_p = adj_ref[pl.ds(base, TP), :]
        Bsub = A_p * gsub_b
        Binter = A_p * ginter_b
        Bcat = jnp.concatenate([Bsub, Binter], axis=0)
        a_p = jnp.sum(Bsub.astype(jnp.float32), axis=1, keepdims=True)
        wv_p = wvcol_ref[pl.ds(base, TP), :]
        pp_p = pprow_ref[0, pl.ds(base, TP)].reshape(TP, 1)
        oh_p = oh_ref[pl.ds(base, TP), :]
        PG_row = jnp.dot(oh_p, predT_ref[...],
                         preferred_element_type=jnp.float32)
        tgt_p = tgt_ref[0, pl.ds(base, TP)].reshape(TP, 1)

        ell_row = (GAMMA - pp_p + PG_row) ** 2
        neq_row = tgt_p != tgt_ref[...]
        wl_row = jnp.where(neq_row, (wv_p * ell_row) * wvrow_ref[...], 0.0)

        for q in range(NP):
            qb = q * TP
            A_cols = adj_ref[:, pl.ds(qb, TP)]
            SC = jnp.dot(Bcat, A_cols, preferred_element_type=jnp.float32)
            S = SC[:TP] + Bsub[:, qb:qb + TP].astype(jnp.float32) \
                * erow_ref[0, pl.ds(qb, TP)][None, :]
            Cm = SC[TP:]
            ratio = (1.0 + a_p - S) / (1.0 + Cm)
            v = 0.5 - 0.5 * jnp.tanh(0.5 * ratio)
            acc = acc + jnp.sum(wl_row[:, qb:qb + TP] * v).reshape(1, 1)
    out_ref[0] = acc


@jax.jit
def kernel(pred, gem, W_sub, W_inter, W_global, target, mask, adj):
    del W_global
    adj_b = adj.astype(jnp.bfloat16)
    tgt = target.astype(jnp.int32).reshape(1, N)
    maskf = mask.astype(jnp.float32).reshape(1, N)

    f32 = jnp.float32
    prep_out = (
        jax.ShapeDtypeStruct((1, N), f32),
        jax.ShapeDtypeStruct((1, N), f32),
        jax.ShapeDtypeStruct((1, N), f32),
        jax.ShapeDtypeStruct((1, N), f32),
        jax.ShapeDtypeStruct((N, 1), f32),
        jax.ShapeDtypeStruct((1, N), f32),
        jax.ShapeDtypeStruct((N, C), f32),
        jax.ShapeDtypeStruct((C, N), f32),
    )
    row = lambda i: (0, i)
    full = lambda i: (0, 0)
    gsub, ginter, erow, wvrow, wvcol, pprow, oh, predT = pl.pallas_call(
        _prep_kernel,
        grid=(PNP,),
        in_specs=[
            pl.BlockSpec((PT, N), lambda i: (i, 0)),
            pl.BlockSpec((PT, PT), lambda i: (i, i)),
            pl.BlockSpec((WSUB := W_sub.shape[0], N), full),
            pl.BlockSpec((WSUB, N), full),
            pl.BlockSpec((N, C), full),
            pl.BlockSpec((1, N), full),
            pl.BlockSpec((1, N), full),
        ],
        out_specs=(
            pl.BlockSpec((1, PT), row),
            pl.BlockSpec((1, PT), row),
            pl.BlockSpec((1, PT), row),
            pl.BlockSpec((1, N), full),
            pl.BlockSpec((N, 1), full),
            pl.BlockSpec((1, N), full),
            pl.BlockSpec((N, C), full),
            pl.BlockSpec((C, N), full),
        ),
        out_shape=prep_out,
    )(gem, adj_b, W_sub, W_inter, pred, tgt, maskf)

    small = (gsub, ginter, wvrow, wvcol, pprow, oh, predT, erow, tgt)
    loss = pl.pallas_call(
        _main_kernel,
        out_shape=jax.ShapeDtypeStruct((1, 1, 1), f32),
    )(adj_b, *small)

    return loss.reshape(1)
